# Initial kernel scaffold; baseline (speedup 1.0000x reference)
import jax
import jax.numpy as jnp
from jax.experimental import pallas as pl


def kernel(loc_data, conf_data, prop_loc_data, prop_conf_data, center_data, priors, act_data, prop_act_data, targets):
    raise NotImplementedError("write your pallas kernel here")


# speedup vs baseline: 113.8459x; 113.8459x over previous
import functools

import jax
import jax.numpy as jnp
import numpy as np
from jax.experimental import pallas as pl
from jax.experimental.pallas import tpu as pltpu

_CLIP = 256.0
_EPS = float(np.finfo(np.float32).eps)
_LB = tuple(v / 256.0 for v in (0.0, 15.0, 30.0, 60.0, 96.0, 256.0))
_RB = tuple(v / 256.0 for v in (30.0, 60.0, 120.0, 240.0, 768.0, 768.0))
_N_TGT = 30


def _levels_to_bounds(lvlf):
    lb = jnp.full_like(lvlf, _LB[0])
    rb = jnp.full_like(lvlf, _RB[0])
    for i in range(1, 6):
        sel = lvlf > (i - 0.5)
        lb = jnp.where(sel, _LB[i], lb)
        rb = jnp.where(sel, _RB[i], rb)
    return lb, rb


def _iou(pl0, pl1, tl0, tl1):
    inter = jnp.minimum(pl0, tl0) + jnp.minimum(pl1, tl1)
    union = (tl0 + tl1) + (pl0 + pl1) - inter
    return inter / jnp.maximum(union, _EPS)


def _losses_from_match(ll, lr, g0, g1, p0, p1, q0, q1, ctr, c, best, bs, be, blab):
    found = best < 2.0
    conf = jnp.where(found, blab, 0.0).astype(jnp.int32)
    lt0 = (c - bs) * _CLIP
    lt1 = (be - c) * _CLIP
    iou = _iou(ll, lr, lt0, lt1)
    pos = conf > 0
    posf = pos.astype(jnp.float32)
    any_pos = jnp.max(posf) > 0.0
    miou = jnp.max(jnp.where(pos, iou, -jnp.inf))
    max_iou = jnp.where(any_pos, miou, 2.0)
    thr = jnp.minimum(jnp.float32(0.5), max_iou)
    prop_conf = jnp.where(iou < thr, 0, conf)
    prop_pos = prop_conf > 0
    ppf = prop_pos.astype(jnp.float32)

    pred_area = ll + lr
    target_area = lt0 + lt1
    inter = jnp.minimum(ll, lt0) + jnp.minimum(lr, lt1)
    union = target_area + pred_area - inter
    ious = inter / jnp.maximum(union, _EPS)
    ac = jnp.maximum(ll, lt0) + jnp.maximum(lr, lt1)
    gious = ious - (ac - union) / jnp.maximum(ac, _EPS)
    loss_l = jnp.sum((1.0 - gious) * posf)

    prop_w = ll + lr
    half_w = 0.5 * prop_w
    plt0 = (lt0 - ll) / half_w
    plt1 = (lt1 - lr) / half_w
    d0 = jnp.abs(p0 - plt0)
    d1 = jnp.abs(p1 - plt1)
    sl0 = jnp.where(d0 < 1.0, 0.5 * d0 * d0, d0 - 0.5)
    sl1 = jnp.where(d1 < 1.0, 0.5 * d1 * d1, d1 - 0.5)
    loss_prop_l = jnp.sum((sl0 + sl1) * ppf)

    cur0 = half_w * p0 + ll
    cur1 = half_w * p1 + lr
    ious_ct = jnp.maximum(_iou(cur0, cur1, lt0, lt1), 0.0)
    bce = jnp.maximum(ctr, 0.0) - ctr * ious_ct + jnp.log1p(jnp.exp(-jnp.abs(ctr)))
    loss_ct = jnp.sum(bce * posf)

    def focal(a, b, tgt_i):
        m = jnp.maximum(a, b)
        za = jnp.exp(a - m)
        zb = jnp.exp(b - m)
        s = za + zb
        is0 = tgt_i == 0
        pt = jnp.where(is0, za, zb) / s
        alpha_t = jnp.where(is0, 0.25, 0.75)
        omp = 1.0 - pt
        return -alpha_t * omp * omp * jnp.log(jnp.maximum(pt, _EPS))

    loss_c = jnp.sum(focal(g0, g1, conf))
    loss_prop_c = jnp.sum(focal(q0, q1, prop_conf))

    np_ = jnp.maximum(jnp.sum(posf), 1.0)
    pnp = jnp.maximum(jnp.sum(ppf), 1.0)
    return loss_l, loss_c, loss_ct, loss_prop_l, loss_prop_c, np_, pnp


def _tc_kernel(ts_ref, te_ref, tl_ref,
               c_ref, lvl_ref,
               ll_ref, lr_ref, g0_ref, g1_ref,
               p0_ref, p1_ref, q0_ref, q1_ref, ctr_ref,
               out_ref):
    c = c_ref[...]
    lvlf = lvl_ref[...]
    lb, rb = _levels_to_bounds(lvlf)

    best = jnp.full_like(c, 2.0)
    bs = jnp.zeros_like(c)
    be = jnp.zeros_like(c)
    blab = jnp.zeros_like(c)
    for m in range(_N_TGT):
        s = ts_ref[0, 0, m]
        e = te_ref[0, 0, m]
        lab = tl_ref[0, 0, m]
        t1 = c - s
        t2 = e - c
        a = t1 + t2
        mn = jnp.minimum(t1, t2)
        mx = jnp.maximum(t1, t2)
        take = (mn >= 0.0) & (mx > lb) & (mx <= rb) & (a < best)
        best = jnp.where(take, a, best)
        bs = jnp.where(take, s, bs)
        be = jnp.where(take, e, be)
        blab = jnp.where(take, lab, blab)

    loss_l, loss_c, loss_ct, loss_prop_l, loss_prop_c, np_, pnp = (
        _losses_from_match(ll_ref[...], lr_ref[...], g0_ref[...], g1_ref[...],
                           p0_ref[...], p1_ref[...], q0_ref[...], q1_ref[...],
                           ctr_ref[...], c, best, bs, be, blab))

    rows = jax.lax.broadcasted_iota(jnp.int32, (8, 128), 0)
    r = jnp.zeros((8, 128), jnp.float32)
    r = jnp.where(rows == 0, loss_l / np_, r)
    r = jnp.where(rows == 1, loss_c / np_, r)
    r = jnp.where(rows == 2, loss_ct / np_, r)
    r = jnp.where(rows == 3, loss_prop_l / pnp, r)
    r = jnp.where(rows == 4, loss_prop_c / pnp, r)
    out_ref[0] = r


@jax.jit
def kernel(loc_data, conf_data, prop_loc_data, prop_conf_data, center_data,
           priors, act_data, prop_act_data, targets):
    B, K, _ = loc_data.shape
    R = K // 128
    c2 = priors[:, 0].reshape(R, 128)
    lvl = priors[:, 1].reshape(R, 128)
    ts = targets[:, :, 0].reshape(B, 1, -1)
    te = targets[:, :, 1].reshape(B, 1, -1)
    tl = targets[:, :, 2].reshape(B, 1, -1)
    ll = loc_data[:, :, 0].reshape(B, R, 128)
    lr = loc_data[:, :, 1].reshape(B, R, 128)
    g0 = conf_data[:, :, 0].reshape(B, R, 128)
    g1 = conf_data[:, :, 1].reshape(B, R, 128)
    p0 = prop_loc_data[:, :, 0].reshape(B, R, 128)
    p1 = prop_loc_data[:, :, 1].reshape(B, R, 128)
    q0 = prop_conf_data[:, :, 0].reshape(B, R, 128)
    q1 = prop_conf_data[:, :, 1].reshape(B, R, 128)
    ctr = center_data[:, :, 0].reshape(B, R, 128)

    smem_spec = pl.BlockSpec((1, 1, _N_TGT), lambda b: (b, 0, 0),
                             memory_space=pltpu.SMEM)
    const_spec = pl.BlockSpec((R, 128), lambda b: (0, 0))
    samp_spec = pl.BlockSpec((1, R, 128), lambda b: (b, 0, 0))

    out = pl.pallas_call(
        _tc_kernel,
        grid=(B,),
        in_specs=[smem_spec, smem_spec, smem_spec,
                  const_spec, const_spec] + [samp_spec] * 9,
        out_specs=pl.BlockSpec((1, 8, 128), lambda b: (b, 0, 0)),
        out_shape=jax.ShapeDtypeStruct((B, 8, 128), jnp.float32),
    )(ts, te, tl, c2, lvl, ll, lr, g0, g1, p0, p1, q0, q1, ctr)
    return jnp.mean(out[:, :5, 0], axis=0)
